# baseline (speedup 1.0000x reference)
import functools

import jax
import jax.numpy as jnp
from jax import lax
from jax.experimental import pallas as pl
from jax.experimental.pallas import tpu as pltpu
from jax.experimental.pallas import tpu_sc as plsc

N, C, H, W = 2, 128, 56, 56
PH = PW = 7
G = 2
SCALE = 0.25
OFFSET = 0.5
K = 1000
KP = 1024
NBIN = PH * PW
NPIX = KP * NBIN
NROWS = N * H * W
NCG = 8
CCG = C // NCG
CP = CCG // 2
TABW = NROWS * CP
NPG = 4
PER_TEC = NPIX // NPG
PCH = 448
NCH = PER_TEC // PCH
LANES = 16
NBLK = PCH // LANES
NJ = 16


def _prep_body(rt_ref, idx_ref, w_ref):
    r = rt_ref[...]
    bk = r[0].astype(jnp.int32)
    sw = r[1] * SCALE - OFFSET
    sh = r[2] * SCALE - OFFSET
    ew = r[3] * SCALE - OFFSET
    eh = r[4] * SCALE - OFFSET
    bin_h = (eh - sh) / PH
    bin_w = (ew - sw) / PW

    pyi = lax.broadcasted_iota(jnp.int32, (PH, G, KP), 0).astype(jnp.float32)
    sv = (lax.broadcasted_iota(jnp.int32, (PH, G, KP), 1).astype(jnp.float32)
          + 0.5) / G
    y = sh[None, None, :] + (pyi + sv) * bin_h[None, None, :]
    my = ((y >= -1.0) & (y <= H)).astype(jnp.float32)
    yc = jnp.clip(y, 0.0, H - 1)
    y0f = jnp.floor(yc)
    y0 = y0f.astype(jnp.int32)
    ly = yc - y0f
    hy = 1.0 - ly
    y1 = jnp.minimum(y0 + 1, H - 1)

    x = sw[None, None, :] + (pyi + sv) * bin_w[None, None, :]
    mx = ((x >= -1.0) & (x <= W)).astype(jnp.float32)
    xc = jnp.clip(x, 0.0, W - 1)
    x0f = jnp.floor(xc)
    x0 = x0f.astype(jnp.int32)
    lx = xc - x0f
    hx = 1.0 - lx
    x1 = jnp.minimum(x0 + 1, W - 1)

    ybase = bk * (H * W)
    inv = 1.0 / (G * G)
    for j in range(NJ):
        sy, sx, cy, cx = (j >> 3) & 1, (j >> 2) & 1, (j >> 1) & 1, j & 1
        yw = (my * (hy if cy == 0 else ly))[:, sy, :]
        yi = (y0 if cy == 0 else y1)[:, sy, :]
        xw = (mx * (hx if cx == 0 else lx))[:, sx, :]
        xi = (x0 if cx == 0 else x1)[:, sx, :]
        wj = (yw[:, None, :] * xw[None, :, :] * inv).reshape(NBIN, KP)
        ij = (ybase[None, None, :] + yi[:, None, :] * W
              + xi[None, :, :]).reshape(NBIN, KP)
        idx_ref[j] = ij
        w_ref[j] = wj


def _sc_body(idx_hbm, w_hbm, tab_hbm, out_hbm, tab_v,
             idx_v0, idx_v1, w_v0, w_v1, out_v0, out_v1,
             si0, si1, so0, so1):
    wid = lax.axis_index("s") * 2 + lax.axis_index("c")
    pg = wid % NPG
    cg = wid // NPG
    pltpu.sync_copy(tab_hbm.at[cg], tab_v)
    base = pg * PER_TEC

    def in_copies(ci, idx_v, w_v, si):
        qb = base + ci * PCH
        return (
            pltpu.make_async_copy(idx_hbm.at[:, pl.ds(qb, PCH)], idx_v, si),
            pltpu.make_async_copy(w_hbm.at[:, pl.ds(qb, PCH)], w_v, si),
        )

    def out_copy(ci, out_v, so):
        qb = base + ci * PCH
        return pltpu.make_async_copy(out_v, out_hbm.at[cg, :, pl.ds(qb, PCH)],
                                     so)

    def compute(idx_v, w_v, out_v):
        @plsc.parallel_loop(0, NBLK, 1, unroll=2)
        def blk(ib):
            pb = ib * LANES
            accs = [None] * CCG
            for j in range(NJ):
                iv = idx_v[j, pl.ds(pb, LANES)]
                wv = w_v[j, pl.ds(pb, LANES)]
                for cp in range(CP):
                    g = plsc.load_gather(tab_v, [iv + (cp * NROWS)])
                    bf = plsc.bitcast(g, jnp.bfloat16)
                    a, b = plsc.unpack(bf, format=plsc.PackFormat.INTERLEAVED)
                    ta = a * wv
                    tb = b * wv
                    c0, c1 = 2 * cp, 2 * cp + 1
                    accs[c0] = ta if accs[c0] is None else accs[c0] + ta
                    accs[c1] = tb if accs[c1] is None else accs[c1] + tb
            for c in range(CCG):
                out_v[c, pl.ds(pb, LANES)] = accs[c]

    bufs = ((idx_v0, w_v0, out_v0, si0, so0),
            (idx_v1, w_v1, out_v1, si1, so1))

    for cp in in_copies(0, idx_v0, w_v0, si0):
        cp.start()

    def pair(ci2, carry):
        for b in range(2):
            idx_v, w_v, out_v, si, so = bufs[b]
            ci = ci2 * 2 + b
            if b == 0:
                for cp in in_copies(ci + 1, idx_v1, w_v1, si1):
                    cp.start()
            else:
                @pl.when(ci2 + 1 < NCH // 2)
                def _():
                    for cp in in_copies(ci + 1, idx_v0, w_v0, si0):
                        cp.start()
            for cp in in_copies(ci, idx_v, w_v, si):
                cp.wait()

            @pl.when(ci2 >= 1)
            def _():
                out_copy(ci - 2, out_v, so).wait()

            compute(idx_v, w_v, out_v)
            out_copy(ci, out_v, so).start()
        return carry

    lax.fori_loop(0, NCH // 2, pair, 0)
    out_copy(NCH - 2, out_v0, so0).wait()
    out_copy(NCH - 1, out_v1, so1).wait()


def kernel(features, rois):
    rt = jnp.transpose(jnp.pad(rois, ((0, KP - K), (0, 0))), (1, 0))
    idx3, w3 = pl.pallas_call(
        _prep_body,
        out_shape=[
            jax.ShapeDtypeStruct((NJ, NBIN, KP), jnp.int32),
            jax.ShapeDtypeStruct((NJ, NBIN, KP), jnp.float32),
        ],
    )(rt)
    idxf = idx3.reshape(NJ, NPIX)
    wf = w3.reshape(NJ, NPIX)

    ftb = jnp.transpose(features, (0, 2, 3, 1)).reshape(
        NROWS, NCG, CP, 2).astype(jnp.bfloat16)
    tab = jnp.transpose(
        lax.bitcast_convert_type(ftb, jnp.int32),
        (1, 2, 0)).reshape(NCG, TABW)

    sc_fn = pl.kernel(
        _sc_body,
        out_type=jax.ShapeDtypeStruct((NCG, CCG, NPIX), jnp.float32),
        mesh=plsc.VectorSubcoreMesh(core_axis_name="c", subcore_axis_name="s"),
        compiler_params=pltpu.CompilerParams(
            use_tc_tiling_on_sc=False, needs_layout_passes=False),
        scratch_types=[
            pltpu.VMEM((TABW,), jnp.int32),
            pltpu.VMEM((NJ, PCH), jnp.int32),
            pltpu.VMEM((NJ, PCH), jnp.int32),
            pltpu.VMEM((NJ, PCH), jnp.float32),
            pltpu.VMEM((NJ, PCH), jnp.float32),
            pltpu.VMEM((CCG, PCH), jnp.float32),
            pltpu.VMEM((CCG, PCH), jnp.float32),
            pltpu.SemaphoreType.DMA,
            pltpu.SemaphoreType.DMA,
            pltpu.SemaphoreType.DMA,
            pltpu.SemaphoreType.DMA,
        ],
    )
    out8 = sc_fn(idxf, wf, tab)

    out = jnp.transpose(
        out8.reshape(NCG, CCG, NBIN, KP)[:, :, :, :K],
        (3, 0, 1, 2)).reshape(K, C, PH, PW)
    return out


# speedup vs baseline: 1.0194x; 1.0194x over previous
import functools

import jax
import jax.numpy as jnp
from jax import lax
from jax.experimental import pallas as pl
from jax.experimental.pallas import tpu as pltpu
from jax.experimental.pallas import tpu_sc as plsc

N, C, H, W = 2, 128, 56, 56
PH = PW = 7
G = 2
SCALE = 0.25
OFFSET = 0.5
K = 1000
KP = 1024
NBIN = PH * PW
NPIX = KP * NBIN
NROWS = N * H * W
NCG = 8
CCG = C // NCG
CP = CCG // 2
TABW = NROWS * CP
NPG = 4
PER_TEC = NPIX // NPG
PCH = 448
NCH = PER_TEC // PCH
LANES = 16
NBLK = PCH // LANES
NJ = 16


def _prep_body(rt_ref, idx_ref, w_ref):
    r = rt_ref[...]
    bk = r[0].astype(jnp.int32)
    sw = r[1] * SCALE - OFFSET
    sh = r[2] * SCALE - OFFSET
    ew = r[3] * SCALE - OFFSET
    eh = r[4] * SCALE - OFFSET
    bin_h = (eh - sh) / PH
    bin_w = (ew - sw) / PW

    pyi = lax.broadcasted_iota(jnp.int32, (PH, G, KP), 0).astype(jnp.float32)
    sv = (lax.broadcasted_iota(jnp.int32, (PH, G, KP), 1).astype(jnp.float32)
          + 0.5) / G
    y = sh[None, None, :] + (pyi + sv) * bin_h[None, None, :]
    my = ((y >= -1.0) & (y <= H)).astype(jnp.float32)
    yc = jnp.clip(y, 0.0, H - 1)
    y0f = jnp.floor(yc)
    y0 = y0f.astype(jnp.int32)
    ly = yc - y0f
    hy = 1.0 - ly
    y1 = jnp.minimum(y0 + 1, H - 1)

    x = sw[None, None, :] + (pyi + sv) * bin_w[None, None, :]
    mx = ((x >= -1.0) & (x <= W)).astype(jnp.float32)
    xc = jnp.clip(x, 0.0, W - 1)
    x0f = jnp.floor(xc)
    x0 = x0f.astype(jnp.int32)
    lx = xc - x0f
    hx = 1.0 - lx
    x1 = jnp.minimum(x0 + 1, W - 1)

    ybase = bk * (H * W)
    inv = 1.0 / (G * G)
    for j in range(NJ):
        sy, sx, cy, cx = (j >> 3) & 1, (j >> 2) & 1, (j >> 1) & 1, j & 1
        yw = (my * (hy if cy == 0 else ly))[:, sy, :]
        yi = (y0 if cy == 0 else y1)[:, sy, :]
        xw = (mx * (hx if cx == 0 else lx))[:, sx, :]
        xi = (x0 if cx == 0 else x1)[:, sx, :]
        wj = (yw[:, None, :] * xw[None, :, :] * inv).reshape(NBIN, KP)
        ij = (ybase[None, None, :] + yi[:, None, :] * W
              + xi[None, :, :]).reshape(NBIN, KP)
        idx_ref[j] = ij
        w_ref[j] = wj


def _sc_body(idx_hbm, w_hbm, feat_hbm, out_hbm, tab_v, stage_v,
             idx_v0, idx_v1, w_v0, w_v1, out_v0, out_v1,
             si0, si1, so0, so1):
    wid = lax.axis_index("s") * 2 + lax.axis_index("c")
    pg = wid % NPG
    cg = wid // NPG
    base = pg * PER_TEC

    hw = H * W
    for cp in range(CP):
        pltpu.sync_copy(feat_hbm.at[:, pl.ds(cg * CCG + 2 * cp, 2), :],
                        stage_v)

        def packrow(ri, c2, cp=cp):
            r = ri * LANES
            for b in range(N):
                a = stage_v[b, 0, pl.ds(r, LANES)]
                bv = stage_v[b, 1, pl.ds(r, LANES)]
                pk = plsc.pack(a, bv, format=plsc.PackFormat.INTERLEAVED)
                tab_v[pl.ds(cp * NROWS + b * hw + r, LANES)] = plsc.bitcast(
                    pk, jnp.int32)
            return c2

        lax.fori_loop(0, hw // LANES, packrow, 0)

    def in_copies(ci, idx_v, w_v, si):
        qb = base + ci * PCH
        return (
            pltpu.make_async_copy(idx_hbm.at[:, pl.ds(qb, PCH)], idx_v, si),
            pltpu.make_async_copy(w_hbm.at[:, pl.ds(qb, PCH)], w_v, si),
        )

    def out_copy(ci, out_v, so):
        qb = base + ci * PCH
        return pltpu.make_async_copy(out_v, out_hbm.at[cg, :, pl.ds(qb, PCH)],
                                     so)

    def compute(idx_v, w_v, out_v):
        def blk(ib, c2):
            pb = ib * LANES
            accs = [None] * CCG
            for j in range(NJ):
                iv = idx_v[j, pl.ds(pb, LANES)]
                wv = w_v[j, pl.ds(pb, LANES)]
                for cp in range(CP):
                    g = plsc.load_gather(tab_v, [iv + (cp * NROWS)])
                    bf = plsc.bitcast(g, jnp.bfloat16)
                    a, b = plsc.unpack(bf, format=plsc.PackFormat.INTERLEAVED)
                    ta = a * wv
                    tb = b * wv
                    c0, c1 = 2 * cp, 2 * cp + 1
                    accs[c0] = ta if accs[c0] is None else accs[c0] + ta
                    accs[c1] = tb if accs[c1] is None else accs[c1] + tb
            for c in range(CCG):
                out_v[c, pl.ds(pb, LANES)] = accs[c]
            return c2

        lax.fori_loop(0, NBLK, blk, 0)

    bufs = ((idx_v0, w_v0, out_v0, si0, so0),
            (idx_v1, w_v1, out_v1, si1, so1))

    for cp in in_copies(0, idx_v0, w_v0, si0):
        cp.start()

    def pair(ci2, carry):
        for b in range(2):
            idx_v, w_v, out_v, si, so = bufs[b]
            ci = ci2 * 2 + b
            if b == 0:
                for cp in in_copies(ci + 1, idx_v1, w_v1, si1):
                    cp.start()
            else:
                @pl.when(ci2 + 1 < NCH // 2)
                def _():
                    for cp in in_copies(ci + 1, idx_v0, w_v0, si0):
                        cp.start()
            for cp in in_copies(ci, idx_v, w_v, si):
                cp.wait()

            @pl.when(ci2 >= 1)
            def _():
                out_copy(ci - 2, out_v, so).wait()

            compute(idx_v, w_v, out_v)
            out_copy(ci, out_v, so).start()
        return carry

    lax.fori_loop(0, NCH // 2, pair, 0)
    out_copy(NCH - 2, out_v0, so0).wait()
    out_copy(NCH - 1, out_v1, so1).wait()


def kernel(features, rois):
    rt = jnp.transpose(jnp.pad(rois, ((0, KP - K), (0, 0))), (1, 0))
    idx3, w3 = pl.pallas_call(
        _prep_body,
        out_shape=[
            jax.ShapeDtypeStruct((NJ, NBIN, KP), jnp.int32),
            jax.ShapeDtypeStruct((NJ, NBIN, KP), jnp.float32),
        ],
    )(rt)
    idxf = idx3.reshape(NJ, NPIX)
    wf = w3.reshape(NJ, NPIX)

    featr = features.reshape(N, C, H * W)

    sc_fn = pl.kernel(
        _sc_body,
        out_type=jax.ShapeDtypeStruct((NCG, CCG, NPIX), jnp.float32),
        mesh=plsc.VectorSubcoreMesh(core_axis_name="c", subcore_axis_name="s"),
        compiler_params=pltpu.CompilerParams(
            use_tc_tiling_on_sc=False, needs_layout_passes=False),
        scratch_types=[
            pltpu.VMEM((TABW,), jnp.int32),
            pltpu.VMEM((N, 2, H * W), jnp.float32),
            pltpu.VMEM((NJ, PCH), jnp.int32),
            pltpu.VMEM((NJ, PCH), jnp.int32),
            pltpu.VMEM((NJ, PCH), jnp.float32),
            pltpu.VMEM((NJ, PCH), jnp.float32),
            pltpu.VMEM((CCG, PCH), jnp.float32),
            pltpu.VMEM((CCG, PCH), jnp.float32),
            pltpu.SemaphoreType.DMA,
            pltpu.SemaphoreType.DMA,
            pltpu.SemaphoreType.DMA,
            pltpu.SemaphoreType.DMA,
        ],
    )
    out8 = sc_fn(idxf, wf, featr)

    out = jnp.transpose(
        out8.reshape(NCG, CCG, NBIN, KP)[:, :, :, :K],
        (3, 0, 1, 2)).reshape(K, C, PH, PW)
    return out


# speedup vs baseline: 1.0221x; 1.0027x over previous
import jax
import jax.numpy as jnp
from jax import lax
from jax.experimental import pallas as pl
from jax.experimental.pallas import tpu as pltpu
from jax.experimental.pallas import tpu_sc as plsc

N, C, H, W = 2, 128, 56, 56
PH = PW = 7
G = 2
SCALE = 0.25
OFFSET = 0.5
K = 1000
KP = 1024
NBIN = PH * PW
NPIX = KP * NBIN
NROWS = N * H * W
NCG = 8
CCG = C // NCG
CP = CCG // 2
TABW = NROWS * CP
NPG = 4
PER_TEC = NPIX // NPG
PCH = 448
NCH = PER_TEC // PCH
LANES = 16
NBLK = PCH // LANES
NJ = 16


def _prep_body(rt_ref, idx_ref, w_ref):
    r = rt_ref[...]
    bk = r[0].astype(jnp.int32)
    sw = r[1] * SCALE - OFFSET
    sh = r[2] * SCALE - OFFSET
    ew = r[3] * SCALE - OFFSET
    eh = r[4] * SCALE - OFFSET
    bin_h = (eh - sh) / PH
    bin_w = (ew - sw) / PW

    pyi = lax.broadcasted_iota(jnp.int32, (PH, G, KP), 0).astype(jnp.float32)
    sv = (lax.broadcasted_iota(jnp.int32, (PH, G, KP), 1).astype(jnp.float32)
          + 0.5) / G
    y = sh[None, None, :] + (pyi + sv) * bin_h[None, None, :]
    my = ((y >= -1.0) & (y <= H)).astype(jnp.float32)
    yc = jnp.clip(y, 0.0, H - 1)
    y0f = jnp.floor(yc)
    y0 = y0f.astype(jnp.int32)
    ly = yc - y0f
    hy = 1.0 - ly
    y1 = jnp.minimum(y0 + 1, H - 1)

    x = sw[None, None, :] + (pyi + sv) * bin_w[None, None, :]
    mx = ((x >= -1.0) & (x <= W)).astype(jnp.float32)
    xc = jnp.clip(x, 0.0, W - 1)
    x0f = jnp.floor(xc)
    x0 = x0f.astype(jnp.int32)
    lx = xc - x0f
    hx = 1.0 - lx
    x1 = jnp.minimum(x0 + 1, W - 1)

    ybase = bk * (H * W)
    inv = 1.0 / (G * G)
    for j in range(NJ):
        sy, sx, cy, cx = (j >> 3) & 1, (j >> 2) & 1, (j >> 1) & 1, j & 1
        yw = (my * (hy if cy == 0 else ly))[:, sy, :]
        yi = (y0 if cy == 0 else y1)[:, sy, :]
        xw = (mx * (hx if cx == 0 else lx))[:, sx, :]
        xi = (x0 if cx == 0 else x1)[:, sx, :]
        wj = (yw[:, None, :] * xw[None, :, :] * inv).reshape(NBIN, KP)
        ij = (ybase[None, None, :] + yi[:, None, :] * W
              + xi[None, :, :]).reshape(NBIN, KP)
        idx_ref[j] = ij
        w_ref[j] = wj


def _sc_body(idx_hbm, w_hbm, feat_hbm, out_hbm, tab_v, stage_v,
             idx_v0, idx_v1, w_v0, w_v1, out_v0, out_v1,
             si0, si1, so0, so1):
    wid = lax.axis_index("s") * 2 + lax.axis_index("c")
    pg = wid % NPG
    cg = wid // NPG
    base = pg * PER_TEC

    hw = H * W
    for cp in range(CP):
        pltpu.sync_copy(feat_hbm.at[:, pl.ds(cg * CCG + 2 * cp, 2), :],
                        stage_v)

        def packrow(ri, c2, cp=cp):
            r = ri * LANES
            for b in range(N):
                a = stage_v[b, 0, pl.ds(r, LANES)]
                bv = stage_v[b, 1, pl.ds(r, LANES)]
                pk = plsc.pack(a, bv, format=plsc.PackFormat.INTERLEAVED)
                tab_v[pl.ds(cp * NROWS + b * hw + r, LANES)] = plsc.bitcast(
                    pk, jnp.int32)
            return c2

        lax.fori_loop(0, hw // LANES, packrow, 0)

    def in_copies(ci, idx_v, w_v, si):
        qb = base + ci * PCH
        return (
            pltpu.make_async_copy(idx_hbm.at[:, pl.ds(qb, PCH)], idx_v, si),
            pltpu.make_async_copy(w_hbm.at[:, pl.ds(qb, PCH)], w_v, si),
        )

    def out_copy(ci, out_v, so):
        qb = base + ci * PCH
        return pltpu.make_async_copy(out_v, out_hbm.at[cg, :, pl.ds(qb, PCH)],
                                     so)

    def compute(idx_v, w_v, out_v):
        def blk(ib, c2):
            pb = ib * LANES
            accs = [None] * CCG
            for j in range(NJ):
                iv = idx_v[j, pl.ds(pb, LANES)]
                wv = w_v[j, pl.ds(pb, LANES)]
                for cp in range(CP):
                    g = plsc.load_gather(tab_v, [iv + (cp * NROWS)])
                    bf = plsc.bitcast(g, jnp.bfloat16)
                    a, b = plsc.unpack(bf, format=plsc.PackFormat.INTERLEAVED)
                    ta = a * wv
                    tb = b * wv
                    c0, c1 = 2 * cp, 2 * cp + 1
                    accs[c0] = ta if accs[c0] is None else accs[c0] + ta
                    accs[c1] = tb if accs[c1] is None else accs[c1] + tb
            for c in range(CCG):
                out_v[c, pl.ds(pb, LANES)] = accs[c]
            return c2

        lax.fori_loop(0, NBLK, blk, 0)

    bufs = ((idx_v0, w_v0, out_v0, si0, so0),
            (idx_v1, w_v1, out_v1, si1, so1))

    for cp in in_copies(0, idx_v0, w_v0, si0):
        cp.start()

    def pair(ci2, carry):
        for b in range(2):
            idx_v, w_v, out_v, si, so = bufs[b]
            ci = ci2 * 2 + b
            if b == 0:
                for cp in in_copies(ci + 1, idx_v1, w_v1, si1):
                    cp.start()
            else:
                @pl.when(ci2 + 1 < NCH // 2)
                def _():
                    for cp in in_copies(ci + 1, idx_v0, w_v0, si0):
                        cp.start()
            for cp in in_copies(ci, idx_v, w_v, si):
                cp.wait()

            @pl.when(ci2 >= 1)
            def _():
                out_copy(ci - 2, out_v, so).wait()

            compute(idx_v, w_v, out_v)
            out_copy(ci, out_v, so).start()
        return carry

    lax.fori_loop(0, NCH // 2, pair, 0)
    out_copy(NCH - 2, out_v0, so0).wait()
    out_copy(NCH - 1, out_v1, so1).wait()


def kernel(features, rois):
    rt = jnp.transpose(jnp.pad(rois, ((0, KP - K), (0, 0))), (1, 0))
    idx3, w3 = pl.pallas_call(
        _prep_body,
        out_shape=[
            jax.ShapeDtypeStruct((NJ, NBIN, KP), jnp.int32),
            jax.ShapeDtypeStruct((NJ, NBIN, KP), jnp.float32),
        ],
    )(rt)
    idxf = idx3.reshape(NJ, NPIX)
    wf = w3.reshape(NJ, NPIX)

    featr = features.reshape(N, C, H * W)

    sc_fn = pl.kernel(
        _sc_body,
        out_type=jax.ShapeDtypeStruct((NCG, CCG, NPIX), jnp.float32),
        mesh=plsc.VectorSubcoreMesh(core_axis_name="c", subcore_axis_name="s"),
        compiler_params=pltpu.CompilerParams(
            use_tc_tiling_on_sc=False, needs_layout_passes=False),
        scratch_types=[
            pltpu.VMEM((TABW,), jnp.int32),
            pltpu.VMEM((N, 2, H * W), jnp.float32),
            pltpu.VMEM((NJ, PCH), jnp.int32),
            pltpu.VMEM((NJ, PCH), jnp.int32),
            pltpu.VMEM((NJ, PCH), jnp.float32),
            pltpu.VMEM((NJ, PCH), jnp.float32),
            pltpu.VMEM((CCG, PCH), jnp.float32),
            pltpu.VMEM((CCG, PCH), jnp.float32),
            pltpu.SemaphoreType.DMA,
            pltpu.SemaphoreType.DMA,
            pltpu.SemaphoreType.DMA,
            pltpu.SemaphoreType.DMA,
        ],
    )
    out8 = sc_fn(idxf, wf, featr)

    out = jnp.transpose(
        out8.reshape(NCG, CCG, NBIN, KP)[:, :, :, :K],
        (3, 0, 1, 2)).reshape(K, C, PH, PW)
    return out
